# baseline (speedup 1.0000x reference)
import functools

import jax
import jax.numpy as jnp
from jax import lax
from jax.experimental import pallas as pl
from jax.experimental.pallas import tpu as pltpu
from jax.experimental.pallas import tpu_sc as plsc

NUM_CL = 100
CP = 128
Q = 1024
D = 256
K = 50000
KB = 400
QB = 256
KC = 400
NCHUNKS = K // KC
NKQ = 4
L = 16
NG = 8


def _dist2_body(x_ref, db_ref, out_ref):
    x = x_ref[...]
    db = db_ref[...]
    a2 = jnp.sum(x * x, axis=1)[None, :]
    b2 = jnp.sum(db * db, axis=1, keepdims=True)
    dt = lax.dot_general(db, x, (((1,), (1,)), ((), ())),
                         preferred_element_type=jnp.float32)
    out_ref[...] = a2 + b2 - 2.0 * dt


def _segmin_body(dist_hbm, labels_hbm, out_hbm,
                 dbuf, lbuf, tmp, stage, shared, *accs):
    cid = lax.axis_index("c")
    sid = lax.axis_index("s")
    kq = sid // NKQ
    cbslot = sid % NKQ
    cb = cbslot * 2 + cid
    col0 = pl.multiple_of(cb * CP, CP)

    inf16 = jnp.full((L,), jnp.inf, dtype=jnp.float32)

    def init_body(i, _):
        for gi in range(NG):
            accs[gi][pl.ds(i * L, L)] = inf16
        return 0
    lax.fori_loop(0, CP, init_body, 0)

    def chunk_body(t, _):
        ci = kq + NKQ * t

        @pl.when(ci < NCHUNKS)
        def _():
            k0 = pl.multiple_of(ci * KC, KC)
            pltpu.sync_copy(labels_hbm.at[pl.ds(k0, KC)], lbuf)
            pltpu.sync_copy(dist_hbm.at[pl.ds(k0, KC), pl.ds(col0, CP)], dbuf)

            def k_body(k16, _):
                kb = k16 * L
                lv = lbuf[pl.ds(kb, L)] * L
                for j in range(L):
                    off = lv[j]
                    for gi in range(NG):
                        d = dbuf[kb + j, pl.ds(gi * L, L)]
                        a = accs[gi][pl.ds(off, L)]
                        accs[gi][pl.ds(off, L)] = jnp.minimum(a, d)
                return 0
            lax.fori_loop(0, KC // L, k_body, 0)
        return 0
    lax.fori_loop(0, (NCHUNKS + NKQ - 1) // NKQ, chunk_body, 0)

    for gi in range(NG):
        pltpu.sync_copy(accs[gi], shared.at[sid, gi])
    plsc.subcore_barrier()

    @pl.when(kq == 0)
    def _():
        for m in range(1, NKQ):
            for gi in range(NG):
                pltpu.sync_copy(shared.at[cbslot + NKQ * m, gi], tmp)

                def m_body(i, _, gi=gi):
                    a = accs[gi][pl.ds(i * L, L)]
                    b = tmp[pl.ds(i * L, L)]
                    accs[gi][pl.ds(i * L, L)] = jnp.minimum(a, b)
                    return 0
                lax.fori_loop(0, CP, m_body, 0)

        magic = jnp.int32(0x5F3759DF)
        for gi in range(NG):
            def c_body(c, _, gi=gi):
                a = accs[gi][pl.ds(c * L, L)]
                d2 = jnp.maximum(a, 0.0)
                bits = lax.bitcast_convert_type(d2, jnp.int32)
                y = lax.bitcast_convert_type(magic - (bits >> 1), jnp.float32)
                y = y * (1.5 - 0.5 * d2 * y * y)
                y = y * (1.5 - 0.5 * d2 * y * y)
                y = y * (1.5 - 0.5 * d2 * y * y)
                dist = d2 * y
                outv = jnp.where(a >= jnp.float32(3e38), -jnp.inf, -dist)
                stage[c, pl.ds(gi * L, L)] = outv
                return 0
            lax.fori_loop(0, CP, c_body, 0)
        pltpu.sync_copy(stage, out_hbm.at[:, pl.ds(col0, CP)])


def kernel(x, embeddings_db, labels_db):
    assert x.shape == (Q, D)
    assert embeddings_db.shape == (K, D)
    labels = labels_db.astype(jnp.int32)

    dist2 = pl.pallas_call(
        _dist2_body,
        grid=(K // KB, Q // QB),
        in_specs=[
            pl.BlockSpec((QB, D), lambda i, j: (j, 0)),
            pl.BlockSpec((KB, D), lambda i, j: (i, 0)),
        ],
        out_specs=pl.BlockSpec((KB, QB), lambda i, j: (i, j)),
        out_shape=jax.ShapeDtypeStruct((K, Q), jnp.float32),
    )(x, embeddings_db)

    mesh = plsc.VectorSubcoreMesh(core_axis_name="c", subcore_axis_name="s")
    segmin = functools.partial(
        pl.kernel,
        out_type=jax.ShapeDtypeStruct((CP, Q), jnp.float32),
        mesh=mesh,
        scratch_types=[
            pltpu.VMEM((KC, CP), jnp.float32),
            pltpu.VMEM((KC,), jnp.int32),
            pltpu.VMEM((CP * L,), jnp.float32),
            pltpu.VMEM((CP, CP), jnp.float32),
            pltpu.VMEM_SHARED((16, NG, CP * L), jnp.float32),
        ] + [pltpu.VMEM((CP * L,), jnp.float32) for _ in range(NG)],
    )(_segmin_body)

    logits_cm = segmin(dist2, labels)
    return logits_cm[:NUM_CL, :].T


# speedup vs baseline: 17.8041x; 17.8041x over previous
import functools

import jax
import jax.numpy as jnp
from jax import lax
from jax.experimental import pallas as pl
from jax.experimental.pallas import tpu as pltpu
from jax.experimental.pallas import tpu_sc as plsc

NUM_CL = 100
CP = 128
Q = 1024
D = 256
K = 50000
KP = 50176
KB = 1024
KC = 448
NCHUNKS = KP // KC
NKQ = 4
L = 16
NG = 8


def _dot_body(x_ref, db_ref, dt_ref, b2_ref, a2_ref):
    i = pl.program_id(0)
    xr = x_ref[...]
    db = db_ref[...]
    dt_ref[...] = lax.dot_general(db, xr * -2.0, (((1,), (1,)), ((), ())),
                                  preferred_element_type=jnp.float32)
    b2_ref[pl.ds(i * KB, KB)] = jnp.sum(db * db, axis=1)

    @pl.when(i == 0)
    def _():
        a2_ref[...] = jnp.sum(xr * xr, axis=1)


def _segmin_body(dt_hbm, labels_hbm, b2_hbm, a2_hbm, out_hbm,
                 dbuf, lbuf, bbuf, abuf, tmp, stage, shared, *accs):
    cid = lax.axis_index("c")
    sid = lax.axis_index("s")
    kq = sid // NKQ
    cbslot = sid % NKQ
    cb = cbslot * 2 + cid
    col0 = pl.multiple_of(cb * CP, CP)

    pltpu.sync_copy(a2_hbm.at[pl.ds(col0, CP)], abuf)

    inf16 = jnp.full((L,), jnp.inf, dtype=jnp.float32)

    def init_body(i, _):
        for gi in range(NG):
            accs[gi][pl.ds(i * L, L)] = inf16
        return 0
    lax.fori_loop(0, CP, init_body, 0)

    def chunk_body(t, _):
        ci = kq + NKQ * t
        k0 = pl.multiple_of(ci * KC, KC)
        pltpu.sync_copy(labels_hbm.at[pl.ds(k0, KC)], lbuf)
        pltpu.sync_copy(b2_hbm.at[pl.ds(k0, KC)], bbuf)
        pltpu.sync_copy(dt_hbm.at[pl.ds(k0, KC), pl.ds(col0, CP)], dbuf)

        def k_body(k16, _):
            kb = k16 * L
            lv = lbuf[pl.ds(kb, L)] * L
            bv = bbuf[pl.ds(kb, L)]
            for j in range(L):
                off = lv[j]
                b2s = bv[j]
                for gi in range(NG):
                    d = dbuf[kb + j, pl.ds(gi * L, L)] + b2s
                    a = accs[gi][pl.ds(off, L)]
                    accs[gi][pl.ds(off, L)] = jnp.minimum(a, d)
            return 0
        lax.fori_loop(0, KC // L, k_body, 0)
        return 0
    lax.fori_loop(0, NCHUNKS // NKQ, chunk_body, 0)

    for gi in range(NG):
        pltpu.sync_copy(accs[gi], shared.at[sid, gi])
    plsc.subcore_barrier()

    @pl.when(kq == 0)
    def _():
        for m in range(1, NKQ):
            for gi in range(NG):
                pltpu.sync_copy(shared.at[cbslot + NKQ * m, gi], tmp)

                def m_body(i, _, gi=gi):
                    a = accs[gi][pl.ds(i * L, L)]
                    b = tmp[pl.ds(i * L, L)]
                    accs[gi][pl.ds(i * L, L)] = jnp.minimum(a, b)
                    return 0
                lax.fori_loop(0, CP, m_body, 0)

        magic = jnp.int32(0x5F3759DF)
        for gi in range(NG):
            a2v = abuf[pl.ds(gi * L, L)]

            def c_body(c, _, gi=gi, a2v=a2v):
                a = accs[gi][pl.ds(c * L, L)] + a2v
                d2 = jnp.maximum(a, 0.0)
                bits = lax.bitcast_convert_type(d2, jnp.int32)
                y = lax.bitcast_convert_type(magic - (bits >> 1), jnp.float32)
                y = y * (1.5 - 0.5 * d2 * y * y)
                y = y * (1.5 - 0.5 * d2 * y * y)
                y = y * (1.5 - 0.5 * d2 * y * y)
                dist = d2 * y
                outv = jnp.where(a >= jnp.float32(3e38), -jnp.inf, -dist)
                stage[c, pl.ds(gi * L, L)] = outv
                return 0
            lax.fori_loop(0, CP, c_body, 0)
        pltpu.sync_copy(stage, out_hbm.at[:, pl.ds(col0, CP)])


def kernel(x, embeddings_db, labels_db):
    assert x.shape == (Q, D)
    assert embeddings_db.shape == (K, D)
    assert labels_db.shape == (K,)
    labels = jnp.concatenate(
        [labels_db.astype(jnp.int32),
         jnp.full((KP - K,), CP - 1, jnp.int32)])
    db_p = jnp.concatenate(
        [embeddings_db, jnp.zeros((KP - K, D), jnp.float32)])

    dt, b2, a2 = pl.pallas_call(
        _dot_body,
        grid=(KP // KB,),
        in_specs=[
            pl.BlockSpec((Q, D), lambda i: (0, 0)),
            pl.BlockSpec((KB, D), lambda i: (i, 0)),
        ],
        out_specs=[
            pl.BlockSpec((KB, Q), lambda i: (i, 0)),
            pl.BlockSpec((KP,), lambda i: (0,)),
            pl.BlockSpec((Q,), lambda i: (0,)),
        ],
        out_shape=[
            jax.ShapeDtypeStruct((KP, Q), jnp.float32),
            jax.ShapeDtypeStruct((KP,), jnp.float32),
            jax.ShapeDtypeStruct((Q,), jnp.float32),
        ],
    )(x, db_p)

    mesh = plsc.VectorSubcoreMesh(core_axis_name="c", subcore_axis_name="s")
    segmin = functools.partial(
        pl.kernel,
        out_type=jax.ShapeDtypeStruct((CP, Q), jnp.float32),
        mesh=mesh,
        scratch_types=[
            pltpu.VMEM((KC, CP), jnp.float32),
            pltpu.VMEM((KC,), jnp.int32),
            pltpu.VMEM((KC,), jnp.float32),
            pltpu.VMEM((CP,), jnp.float32),
            pltpu.VMEM((CP * L,), jnp.float32),
            pltpu.VMEM((CP, CP), jnp.float32),
            pltpu.VMEM_SHARED((16, NG, CP * L), jnp.float32),
        ] + [pltpu.VMEM((CP * L,), jnp.float32) for _ in range(NG)],
    )(_segmin_body)

    logits_cm = segmin(dt, labels, b2, a2)
    return logits_cm[:NUM_CL, :].T


# speedup vs baseline: 25.5380x; 1.4344x over previous
import functools

import jax
import jax.numpy as jnp
from jax import lax
from jax.experimental import pallas as pl
from jax.experimental.pallas import tpu as pltpu
from jax.experimental.pallas import tpu_sc as plsc

NUM_CL = 100
CPA = 104
CP = 128
Q = 1024
D = 256
K = 50000
KP = 50176
KB = 1024
KC = 128
NCHUNKS = KP // KC
NKQ = 4
L = 16
LB = 32
NG = 8
CENTER = 256.0


def _dot_body(x_ref, db_ref, dt_ref, a2_ref):
    i = pl.program_id(0)
    xr = x_ref[...]
    db = db_ref[...]
    ones = jnp.ones((D, 1), jnp.float32)
    b2c = lax.dot_general(db * db, ones, (((1,), (0,)), ((), ())),
                          preferred_element_type=jnp.float32)
    dtf = lax.dot_general(db.astype(jnp.bfloat16),
                          (xr * -2.0).astype(jnp.bfloat16),
                          (((1,), (1,)), ((), ())),
                          preferred_element_type=jnp.float32)
    dt_ref[...] = dtf + (b2c - CENTER)

    @pl.when(i == 0)
    def _():
        a2_ref[...] = jnp.sum(xr * xr, axis=1)


def _segmin_body(dt_hbm, labels_hbm, out_hbm,
                 dbuf0, dbuf1, lbuf0, lbuf1, stage, sem0, sem1, *accs):
    cid = lax.axis_index("c")
    sid = lax.axis_index("s")
    kq = sid // NKQ
    cbslot = sid % NKQ
    cb = cbslot * 2 + cid
    col0 = pl.multiple_of(cb * CP, CP)

    dbufs, lbufs, sems = (dbuf0, dbuf1), (lbuf0, lbuf1), (sem0, sem1)

    def _copies(ci, ph):
        k0 = pl.multiple_of(ci * KC, KC)
        return (
            pltpu.make_async_copy(
                dt_hbm.at[pl.ds(k0, KC), pl.ds(col0, CP)], dbufs[ph], sems[ph]),
            pltpu.make_async_copy(
                labels_hbm.at[pl.ds(k0, KC)], lbufs[ph], sems[ph]),
        )

    def issue(ci, ph):
        for c in _copies(ci, ph):
            c.start()

    def drain(ci, ph):
        for c in _copies(ci, ph):
            c.wait()

    infv = jnp.full((L,), jnp.inf, dtype=jnp.float32)

    def init_body(c, _):
        for gi in range(NG):
            accs[gi][pl.ds(c * L, L)] = infv
        return 0
    lax.fori_loop(0, CPA, init_body, 0)

    issue(kq, 0)
    issue(kq + NKQ, 1)

    def chunk_pair(t2, _):
        for ph in range(2):
            t = t2 * 2 + ph
            ci = kq + NKQ * t
            drain(ci, ph)
            dbuf, lbuf = dbufs[ph], lbufs[ph]

            def k_body(k16, _, dbuf=dbuf, lbuf=lbuf):
                kb = k16 * L
                lv = lbuf[pl.ds(kb, L)] * L
                for j in range(L):
                    off = lv[j]
                    for gi in range(NG):
                        d = dbuf[kb + j, pl.ds(gi * L, L)]
                        a = accs[gi][pl.ds(off, L)]
                        accs[gi][pl.ds(off, L)] = jnp.minimum(a, d)
                return 0
            lax.fori_loop(0, KC // L, k_body, 0)

            @pl.when(t < (NCHUNKS // NKQ) - 2)
            def _():
                issue(ci + 2 * NKQ, ph)
        return 0
    lax.fori_loop(0, NCHUNKS // NKQ // 2, chunk_pair, 0)

    def fin_body(c, _):
        for gi in range(NG):
            stage[c, pl.ds(gi * L, L)] = accs[gi][pl.ds(c * L, L)]
        return 0
    lax.fori_loop(0, CPA, fin_body, 0)
    pltpu.sync_copy(stage, out_hbm.at[kq, :, pl.ds(col0, CP)])


def _epi_body(part_ref, a2_ref, out_ref):
    m = part_ref[0]
    for m2 in range(1, NKQ):
        m = jnp.minimum(m, part_ref[m2])
    d2 = jnp.maximum(a2_ref[...][None, :] + (m + CENTER), 0.0)
    out_ref[...] = -jnp.sqrt(d2)


def kernel(x, embeddings_db, labels_db):
    assert x.shape == (Q, D)
    assert embeddings_db.shape == (K, D)
    assert labels_db.shape == (K,)
    labels = jnp.concatenate(
        [labels_db.astype(jnp.int32),
         jnp.full((KP - K,), NUM_CL, jnp.int32)])
    db_p = jnp.concatenate(
        [embeddings_db, jnp.zeros((KP - K, D), jnp.float32)])

    dt, a2 = pl.pallas_call(
        _dot_body,
        grid=(KP // KB,),
        in_specs=[
            pl.BlockSpec((Q, D), lambda i: (0, 0)),
            pl.BlockSpec((KB, D), lambda i: (i, 0)),
        ],
        out_specs=[
            pl.BlockSpec((KB, Q), lambda i: (i, 0)),
            pl.BlockSpec((Q,), lambda i: (0,)),
        ],
        out_shape=[
            jax.ShapeDtypeStruct((KP, Q), jnp.float32),
            jax.ShapeDtypeStruct((Q,), jnp.float32),
        ],
    )(x, db_p)

    mesh = plsc.VectorSubcoreMesh(core_axis_name="c", subcore_axis_name="s")
    segmin = functools.partial(
        pl.kernel,
        out_type=jax.ShapeDtypeStruct((NKQ, CPA, Q), jnp.float32),
        mesh=mesh,
        scratch_types=[
            pltpu.VMEM((KC, CP), jnp.float32),
            pltpu.VMEM((KC, CP), jnp.float32),
            pltpu.VMEM((KC,), jnp.int32),
            pltpu.VMEM((KC,), jnp.int32),
            pltpu.VMEM((CPA, CP), jnp.float32),
            pltpu.SemaphoreType.DMA,
            pltpu.SemaphoreType.DMA,
        ] + [pltpu.VMEM((CPA * L,), jnp.float32) for _ in range(NG)],
    )(_segmin_body)

    part = segmin(dt, labels)

    QE = 128
    logits_cm = pl.pallas_call(
        _epi_body,
        grid=(Q // QE,),
        in_specs=[
            pl.BlockSpec((NKQ, CPA, QE), lambda j: (0, 0, j)),
            pl.BlockSpec((QE,), lambda j: (j,)),
        ],
        out_specs=pl.BlockSpec((CPA, QE), lambda j: (0, j)),
        out_shape=jax.ShapeDtypeStruct((CPA, Q), jnp.float32),
    )(part, a2)

    return logits_cm[:NUM_CL, :].T
